# baseline (speedup 1.0000x reference)
import functools

import jax
import jax.numpy as jnp
from jax import lax
from jax.experimental import pallas as pl
from jax.experimental.pallas import tpu as pltpu
from jax.experimental.pallas import tpu_sc as plsc

N, R, A, F, T = 10000, 5, 8, 64, 16
NRA = N * R * A
NC, NS = 2, 16
NW = NC * NS
CHUNK = 160
NCHUNKS = NRA // CHUNK
CPW = (NCHUNKS + NW - 1) // NW
NG = CHUNK // 16


def _sc_interp(table, bc_flat):
    mesh = plsc.VectorSubcoreMesh(core_axis_name="c", subcore_axis_name="s")

    @functools.partial(
        pl.kernel,
        out_type=jax.ShapeDtypeStruct((NRA, F), jnp.float32),
        mesh=mesh,
        compiler_params=pltpu.CompilerParams(use_tc_tiling_on_sc=False,
                                             needs_layout_passes=False),
        scratch_types=[
            pltpu.VMEM((6 * CHUNK,), jnp.float32),
            pltpu.VMEM((6 * CHUNK,), jnp.float32),
            pltpu.VMEM((3 * CHUNK,), jnp.int32),
            pltpu.VMEM((3 * CHUNK,), jnp.int32),
            pltpu.VMEM((3 * CHUNK,), jnp.float32),
            pltpu.VMEM((3 * CHUNK,), jnp.float32),
            pltpu.VMEM((3 * CHUNK, F), jnp.float32),
            pltpu.VMEM((3 * CHUNK, F), jnp.float32),
            pltpu.VMEM((CHUNK, F), jnp.float32),
            pltpu.SemaphoreType.DMA,
            pltpu.SemaphoreType.DMA,
        ],
    )
    def k(table_hbm, bc_hbm, out_hbm,
          bc0, bc1, x0, x1, w0, w1, rows0, rows1, acc, gs0, gs1):
        wid = lax.axis_index("s") * NC + lax.axis_index("c")
        iota16 = lax.iota(jnp.int32, 16)
        bcs = (bc0, bc1)
        xs = (x0, x1)
        ws = (w0, w1)
        rowss = (rows0, rows1)
        gss = (gs0, gs1)

        def prep_and_fire(ci, b):
            bcv, xv, wv, rv, gs = bcs[b], xs[b], ws[b], rowss[b], gss[b]
            pltpu.sync_copy(bc_hbm.at[pl.ds(ci * (6 * CHUNK), 6 * CHUNK)], bcv)

            def pg(g, c):
                base_v = 6 * (g * 16) + 6 * iota16
                for t in range(3):
                    fi = plsc.load_gather(bcv, [base_v + (2 * t)])
                    xv[pl.ds(t * CHUNK + g * 16, 16)] = fi.astype(jnp.int32)
                    wt = plsc.load_gather(bcv, [base_v + (2 * t + 1)])
                    wv[pl.ds(t * CHUNK + g * 16, 16)] = wt
                return c

            lax.fori_loop(0, NG, pg, 0)
            for t in range(3):
                sl = pl.ds(t * CHUNK, CHUNK)
                pltpu.async_copy(table_hbm.at[xv.at[sl]], rv.at[sl], gs)

        def slot(kk, b):
            ci = wid + kk * NW

            @pl.when(ci < NCHUNKS)
            def _():
                rv, wv, gs = rowss[b], ws[b], gss[b]
                pltpu.make_async_copy(out_hbm.at[pl.ds(0, 3 * CHUNK)], rv, gs).wait()
                cn = ci + NW

                @pl.when(cn < NCHUNKS)
                def _():
                    prep_and_fire(cn, 1 - b)

                def group_body(g, c2):
                    gb = g * 16
                    wv0 = wv[pl.ds(gb, 16)]
                    wv1 = wv[pl.ds(CHUNK + gb, 16)]
                    wv2 = wv[pl.ds(2 * CHUNK + gb, 16)]
                    for j in range(16):
                        i = gb + j
                        for cb in range(F // 16):
                            sl = pl.ds(cb * 16, 16)
                            acc[i, sl] = (wv0[j] * rv[i, sl]
                                          + wv1[j] * rv[CHUNK + i, sl]
                                          + wv2[j] * rv[2 * CHUNK + i, sl])
                    return c2

                lax.fori_loop(0, NG, group_body, 0)
                pltpu.sync_copy(acc, out_hbm.at[pl.ds(ci * CHUNK, CHUNK)])

        prep_and_fire(wid, 0)

        def body(kk2, c):
            slot(kk2 * 2, 0)
            slot(kk2 * 2 + 1, 1)
            return c

        lax.fori_loop(0, (CPW + 1) // 2, body, 0)

    return k(table, bc_flat)


def _tc_matmul_bias_relu(a, b, bias_row):
    n, kdim = a.shape
    m = b.shape[1]
    bn = 1000

    def mmk(a_ref, b_ref, bias_ref, o_ref):
        o = jnp.dot(a_ref[...], b_ref[...], preferred_element_type=jnp.float32)
        o_ref[...] = jnp.maximum(o + bias_ref[...], 0.0)

    return pl.pallas_call(
        mmk,
        grid=(n // bn,),
        in_specs=[
            pl.BlockSpec((bn, kdim), lambda i: (i, 0)),
            pl.BlockSpec((kdim, m), lambda i: (0, 0)),
            pl.BlockSpec((1, m), lambda i: (0, 0)),
        ],
        out_specs=pl.BlockSpec((bn, m), lambda i: (i, 0)),
        out_shape=jax.ShapeDtypeStruct((n, m), jnp.float32),
    )(a, b, bias_row)


def kernel(mesh_signal, bary_coordinates, template_weights, bias):
    bc_flat = bary_coordinates.reshape(NRA * 6)

    interp = _sc_interp(mesh_signal, bc_flat)

    tw0 = template_weights[:, 0].reshape(T, F, R, A)
    rot_idx = (jnp.arange(A)[None, :] + jnp.arange(A)[:, None]) % A
    twr = tw0[:, :, :, rot_idx]
    wbig = jnp.transpose(twr, (2, 4, 1, 3, 0)).reshape(R * A * F, A * T)
    bias_row = jnp.tile(bias[:, 0], A)[None, :]

    out = _tc_matmul_bias_relu(interp.reshape(N, R * A * F), wbig, bias_row)
    return out.reshape(N, A, T)


# speedup vs baseline: 5.3211x; 5.3211x over previous
import functools

import jax
import jax.numpy as jnp
from jax import lax
from jax.experimental import pallas as pl
from jax.experimental.pallas import tpu as pltpu
from jax.experimental.pallas import tpu_sc as plsc

N, R, A, F, T = 10000, 5, 8, 64, 16
Q = R * A
NRA = N * Q
NC, NS = 2, 16
NW = NC * NS
VBLK = 80
NBLK = N // VBLK
BPW = (NBLK + NW - 1) // NW
VCH = 4
CHUNK = VCH * Q
CPB = VBLK // VCH
NG = CHUNK // 16


def _sc_interp(table, i0, i1, i2, w0a, w1a, w2a):
    mesh = plsc.VectorSubcoreMesh(core_axis_name="c", subcore_axis_name="s")

    @functools.partial(
        pl.kernel,
        out_type=jax.ShapeDtypeStruct((NRA, F), jnp.float32),
        mesh=mesh,
        compiler_params=pltpu.CompilerParams(use_tc_tiling_on_sc=False,
                                             needs_layout_passes=False),
        scratch_types=[
            pltpu.VMEM((3, Q, VBLK), jnp.int32),
            pltpu.VMEM((3, Q, VBLK), jnp.float32),
            pltpu.VMEM((3 * CHUNK,), jnp.int32),
            pltpu.VMEM((3 * CHUNK,), jnp.int32),
            pltpu.VMEM((3 * CHUNK,), jnp.float32),
            pltpu.VMEM((3 * CHUNK,), jnp.float32),
            pltpu.VMEM((3 * CHUNK, F), jnp.float32),
            pltpu.VMEM((3 * CHUNK, F), jnp.float32),
            pltpu.VMEM((CHUNK, F), jnp.float32),
            pltpu.SemaphoreType.DMA,
            pltpu.SemaphoreType.DMA,
        ],
    )
    def k(table_hbm, i0_hbm, i1_hbm, i2_hbm, w0_hbm, w1_hbm, w2_hbm, out_hbm,
          pidx, pwts, x0, x1, w0, w1, rows0, rows1, acc, gs0, gs1):
        wid = lax.axis_index("s") * NC + lax.axis_index("c")
        iota16 = lax.iota(jnp.int32, 16)
        xs = (x0, x1)
        ws = (w0, w1)
        rowss = (rows0, rows1)
        gss = (gs0, gs1)
        ihbms = (i0_hbm, i1_hbm, i2_hbm)
        whbms = (w0_hbm, w1_hbm, w2_hbm)

        def prep_and_fire(cc, b):
            xv, wv, rv, gs = xs[b], ws[b], rowss[b], gss[b]

            def pg(g, c):
                lvec = cc * CHUNK + g * 16 + iota16
                vv = lvec // Q
                qq = lvec - vv * Q
                for t in range(3):
                    fi = plsc.load_gather(pidx.at[t], [qq, vv])
                    xv[pl.ds(t * CHUNK + g * 16, 16)] = fi
                    wt = plsc.load_gather(pwts.at[t], [qq, vv])
                    wv[pl.ds(t * CHUNK + g * 16, 16)] = wt
                return c

            lax.fori_loop(0, NG, pg, 0)
            for t in range(3):
                sl = pl.ds(t * CHUNK, CHUNK)
                pltpu.async_copy(table_hbm.at[xv.at[sl]], rv.at[sl], gs)

        def block_body(kb, carry):
            bi = wid + kb * NW

            @pl.when(bi < NBLK)
            def _():
                v0 = bi * VBLK
                for t in range(3):
                    pltpu.sync_copy(ihbms[t].at[:, pl.ds(v0, VBLK)], pidx.at[t])
                    pltpu.sync_copy(whbms[t].at[:, pl.ds(v0, VBLK)], pwts.at[t])
                prep_and_fire(0, 0)

                def slot(cc, b):
                    rv, wv, gs = rowss[b], ws[b], gss[b]
                    pltpu.make_async_copy(
                        out_hbm.at[pl.ds(0, 3 * CHUNK)], rv, gs).wait()

                    @pl.when(cc < CPB - 1)
                    def _():
                        prep_and_fire(cc + 1, 1 - b)

                    def group_body(g, c2):
                        gb = g * 16
                        wv0 = wv[pl.ds(gb, 16)]
                        wv1 = wv[pl.ds(CHUNK + gb, 16)]
                        wv2 = wv[pl.ds(2 * CHUNK + gb, 16)]
                        for j in range(16):
                            i = gb + j
                            for cb in range(F // 16):
                                sl = pl.ds(cb * 16, 16)
                                acc[i, sl] = (wv0[j] * rv[i, sl]
                                              + wv1[j] * rv[CHUNK + i, sl]
                                              + wv2[j] * rv[2 * CHUNK + i, sl])
                        return c2

                    lax.fori_loop(0, NG, group_body, 0)
                    base = bi * (VBLK * Q) + cc * CHUNK
                    pltpu.sync_copy(acc, out_hbm.at[pl.ds(base, CHUNK)])

                def pair(cc2, c):
                    slot(cc2 * 2, 0)
                    slot(cc2 * 2 + 1, 1)
                    return c

                lax.fori_loop(0, CPB // 2, pair, 0)

            return carry

        lax.fori_loop(0, BPW, block_body, 0)

    return k(table, i0, i1, i2, w0a, w1a, w2a)


def _tc_matmul_bias_relu(a, b, bias_row):
    n, kdim = a.shape
    m = b.shape[1]
    bn = 1000

    def mmk(a_ref, b_ref, bias_ref, o_ref):
        o = jnp.dot(a_ref[...], b_ref[...], preferred_element_type=jnp.float32)
        o_ref[...] = jnp.maximum(o + bias_ref[...], 0.0)

    return pl.pallas_call(
        mmk,
        grid=(n // bn,),
        in_specs=[
            pl.BlockSpec((bn, kdim), lambda i: (i, 0)),
            pl.BlockSpec((kdim, m), lambda i: (0, 0)),
            pl.BlockSpec((1, m), lambda i: (0, 0)),
        ],
        out_specs=pl.BlockSpec((bn, m), lambda i: (i, 0)),
        out_shape=jax.ShapeDtypeStruct((n, m), jnp.float32),
    )(a, b, bias_row)


def kernel(mesh_signal, bary_coordinates, template_weights, bias):
    def plane(t, c):
        p = bary_coordinates[:, :, :, t, c].transpose(1, 2, 0).reshape(Q, N)
        return p.astype(jnp.int32) if c == 0 else p

    i0, i1, i2 = plane(0, 0), plane(1, 0), plane(2, 0)
    w0a, w1a, w2a = plane(0, 1), plane(1, 1), plane(2, 1)

    interp = _sc_interp(mesh_signal, i0, i1, i2, w0a, w1a, w2a)

    tw0 = template_weights[:, 0].reshape(T, F, R, A)
    rot_idx = (jnp.arange(A)[None, :] + jnp.arange(A)[:, None]) % A
    twr = tw0[:, :, :, rot_idx]
    wbig = jnp.transpose(twr, (2, 4, 1, 3, 0)).reshape(R * A * F, A * T)
    bias_row = jnp.tile(bias[:, 0], A)[None, :]

    out = _tc_matmul_bias_relu(interp.reshape(N, R * A * F), wbig, bias_row)
    return out.reshape(N, A, T)
